# baseline (speedup 1.0000x reference)
import jax
import jax.numpy as jnp
from jax.experimental import pallas as pl

E = 64
H = 768
I = 3072


def _ffn_block_kernel(x_ref, w1_ref, b1_ref, w2_ref, b2_ref, o_ref):
    x = x_ref[0]
    h = jnp.dot(x, w1_ref[0], preferred_element_type=jnp.float32)
    h = h + b1_ref[0]
    o = jnp.dot(h, w2_ref[0], preferred_element_type=jnp.float32)
    o_ref[0] = o + b2_ref[0]


def kernel(hidden_states, W1, b1, W2, b2):
    Bb, Ss, Hh = hidden_states.shape
    Ee = W1.shape[0]
    per = (Bb * Ss) // Ee
    x = hidden_states.reshape(Ee, per, Hh)
    b1r = b1.reshape(Ee, 1, I)
    b2r = b2.reshape(Ee, 1, Hh)

    out = pl.pallas_call(
        _ffn_block_kernel,
        grid=(Ee,),
        in_specs=[
            pl.BlockSpec((1, per, Hh), lambda e: (e, 0, 0)),
            pl.BlockSpec((1, Hh, I), lambda e: (e, 0, 0)),
            pl.BlockSpec((1, 1, I), lambda e: (e, 0, 0)),
            pl.BlockSpec((1, I, Hh), lambda e: (e, 0, 0)),
            pl.BlockSpec((1, 1, Hh), lambda e: (e, 0, 0)),
        ],
        out_specs=pl.BlockSpec((1, per, Hh), lambda e: (e, 0, 0)),
        out_shape=jax.ShapeDtypeStruct((Ee, per, Hh), jnp.float32),
    )(x, W1, b1r, W2, b2r)
    return out.reshape(Bb, Ss, Hh)


# speedup vs baseline: 1.0361x; 1.0361x over previous
import jax
import jax.numpy as jnp
from jax.experimental import pallas as pl
from jax.experimental.pallas import tpu as pltpu

E = 64
H = 768
I = 3072
I_BLK = 1024


def _ffn_block_kernel(x_ref, w1_ref, b1_ref, w2_ref, b2_ref, o_ref):
    k = pl.program_id(1)
    x = x_ref[0]
    h = jnp.dot(x, w1_ref[0], preferred_element_type=jnp.float32)
    h = h + b1_ref[0]
    o = jnp.dot(h, w2_ref[0], preferred_element_type=jnp.float32)

    @pl.when(k == 0)
    def _init():
        o_ref[0] = o + b2_ref[0]

    @pl.when(k != 0)
    def _acc():
        o_ref[0] += o


def kernel(hidden_states, W1, b1, W2, b2):
    Bb, Ss, Hh = hidden_states.shape
    Ee = W1.shape[0]
    per = (Bb * Ss) // Ee
    K = I // I_BLK
    x = hidden_states.reshape(Ee, per, Hh)
    b1r = b1.reshape(Ee, 1, I)
    b2r = b2.reshape(Ee, 1, Hh)

    out = pl.pallas_call(
        _ffn_block_kernel,
        grid=(Ee, K),
        in_specs=[
            pl.BlockSpec((1, per, Hh), lambda e, k: (e, 0, 0)),
            pl.BlockSpec((1, Hh, I_BLK), lambda e, k: (e, 0, k)),
            pl.BlockSpec((1, 1, I_BLK), lambda e, k: (e, 0, k)),
            pl.BlockSpec((1, I_BLK, Hh), lambda e, k: (e, k, 0)),
            pl.BlockSpec((1, 1, Hh), lambda e, k: (e, 0, 0)),
        ],
        out_specs=pl.BlockSpec((1, per, Hh), lambda e, k: (e, 0, 0)),
        out_shape=jax.ShapeDtypeStruct((Ee, per, Hh), jnp.float32),
        compiler_params=pltpu.CompilerParams(
            dimension_semantics=("parallel", "arbitrary"),
        ),
    )(x, W1, b1r, W2, b2r)
    return out.reshape(Bb, Ss, Hh)
